# Initial kernel scaffold; baseline (speedup 1.0000x reference)
import jax
import jax.numpy as jnp
from jax.experimental import pallas as pl


def kernel(categories, ce):
    raise NotImplementedError("write your pallas kernel here")


# speedup vs baseline: 3.2690x; 3.2690x over previous
import functools

import jax
import jax.numpy as jnp
from jax import lax
from jax.experimental import pallas as pl
from jax.experimental.pallas import tpu as pltpu
from jax.experimental.pallas import tpu_sc as plsc


def _gather_kernel(N, D, NW, per_w, C, nch):
    mesh = plsc.VectorSubcoreMesh(core_axis_name="c", subcore_axis_name="s")

    @functools.partial(
        pl.kernel,
        mesh=mesh,
        out_type=jax.ShapeDtypeStruct((N, D), jnp.float32),
        scratch_types=[
            pltpu.VMEM((nch, C), jnp.int32),
            pltpu.VMEM((2, C, D), jnp.float32),
            pltpu.SemaphoreType.DMA,
        ],
    )
    def k(idx_hbm, table_hbm, out_hbm, idx_v, rows_v, gsem):
        wid = lax.axis_index("s") * 2 + lax.axis_index("c")
        base = wid * per_w
        pltpu.sync_copy(idx_hbm.at[wid], idx_v)

        def body(g, _):
            pltpu.async_copy(table_hbm.at[idx_v.at[g]], rows_v.at[0], gsem).wait()
            pltpu.sync_copy(rows_v.at[0], out_hbm.at[pl.ds(base + g * C, C)])
            return 0

        lax.fori_loop(0, nch, body, 0)

    return k


def kernel(categories, ce):
    B, S = categories.shape
    V, D = ce.shape
    N = B * S
    NW = 32
    per_w = N // NW
    C = 128
    nch = per_w // C
    idx3 = categories.reshape(NW, nch, C)
    out = _gather_kernel(N, D, NW, per_w, C, nch)(idx3, ce)
    return out.reshape(B, S, D)
